# baseline (speedup 1.0000x reference)
import jax
import jax.numpy as jnp
from jax.experimental import pallas as pl
from jax.experimental.pallas import tpu as pltpu

_NMS_THR = 0.45
_PRE = 0.005
_B, _N, _C = 8, 5000, 4
_P = _B * _C
_T = 128
_NPAD = 5120
_NB = _NPAD // _T
_CH = 32
_BIG = 1e30


def _suppress_kernel(x1r, y1r, x2r, y2r, aar, ssr, keepr,
                     tx1, ty1, tx2, ty2, tas):
    keepr[:, :] = jnp.ones((_P, _NPAD), jnp.float32)
    lane = jax.lax.broadcasted_iota(jnp.int32, (_P, _T), 1)

    def outer(k, carry):
        base = k * _T
        bx1 = x1r[:, pl.ds(base, _T)]
        by1 = y1r[:, pl.ds(base, _T)]
        bx2 = x2r[:, pl.ds(base, _T)]
        by2 = y2r[:, pl.ds(base, _T)]
        baa = aar[:, pl.ds(base, _T)]
        bss = ssr[:, pl.ds(base, _T)]

        tr = lambda v: jnp.transpose(v, (1, 0))[:, :, None]
        tx1[...] = tr(bx1)
        ty1[...] = tr(by1)
        tx2[...] = tr(bx2)
        ty2[...] = tr(by2)
        tas[...] = tr(jnp.where(bss > 0.0, baa, _BIG))

        def intra(i, carry1):
            ax1 = tx1[i]
            ay1 = ty1[i]
            ax2 = tx2[i]
            ay2 = ty2[i]
            aas = tas[i]
            bkeep = keepr[:, pl.ds(base, _T)]
            ohf = jnp.where(lane == i, 1.0, 0.0)
            ki = jnp.sum(bkeep * ohf, axis=1, keepdims=True)
            akk = jnp.where(ki > 0.0, aas, _BIG)
            iw = jnp.maximum(jnp.minimum(ax2, bx2) - jnp.maximum(ax1, bx1), 0.0)
            ih = jnp.minimum(ay2, by2) - jnp.maximum(ay1, by1)
            sup = jnp.logical_and(iw * ih > (akk + baa), lane > i)
            keepr[:, pl.ds(base, _T)] = jnp.where(sup, 0.0, bkeep)
            return carry1

        jax.lax.fori_loop(0, _T, intra, 0)
        bkeep = keepr[:, pl.ds(base, _T)]

        kaa = jnp.where(jnp.logical_and(bkeep > 0.0, bss > 0.0), baa, _BIG)
        ex1 = bx1[:, :, None]
        ey1 = by1[:, :, None]
        ex2 = bx2[:, :, None]
        ey2 = by2[:, :, None]
        eaa = kaa[:, :, None]

        def over_m(m, carry2):
            mb = m * _T
            mx1 = x1r[:, pl.ds(mb, _T)][:, None, :]
            my1 = y1r[:, pl.ds(mb, _T)][:, None, :]
            mx2 = x2r[:, pl.ds(mb, _T)][:, None, :]
            my2 = y2r[:, pl.ds(mb, _T)][:, None, :]
            maa = aar[:, pl.ds(mb, _T)][:, None, :]
            dj = jnp.full((_P, _T), -1.0, jnp.float32)
            for c in range(0, _T, _CH):
                cs = lambda v: jax.lax.slice_in_dim(v, c, c + _CH, axis=1)
                iw = jnp.maximum(
                    jnp.minimum(cs(ex2), mx2) - jnp.maximum(cs(ex1), mx1), 0.0)
                ih = jnp.minimum(cs(ey2), my2) - jnp.maximum(cs(ey1), my1)
                d = iw * ih - (cs(eaa) + maa)
                dj = jnp.maximum(dj, jnp.max(d, axis=1))
            mk = keepr[:, pl.ds(mb, _T)]
            keepr[:, pl.ds(mb, _T)] = jnp.where(dj > 0.0, 0.0, mk)
            return carry2

        jax.lax.fori_loop(k + 1, _NB, over_m, 0)
        return carry

    jax.lax.fori_loop(0, _NB, outer, 0)


def _run_suppress(x1, y1, x2, y2, aa, ss, interpret=False):
    return pl.pallas_call(
        _suppress_kernel,
        out_shape=jax.ShapeDtypeStruct((_P, _NPAD), jnp.float32),
        scratch_shapes=[pltpu.VMEM((_T, _P, 1), jnp.float32)] * 5,
        interpret=interpret,
    )(x1, y1, x2, y2, aa, ss)


def kernel(bbs, conf):
    s = jnp.where(conf > _PRE, conf, 0.0).reshape(_P, _N)
    order = jnp.argsort(-s, axis=-1)
    ss = jnp.take_along_axis(s, order, axis=-1)
    bx = jnp.broadcast_to(bbs[:, None], (_B, _C, _N, 4)).reshape(_P, _N, 4)
    sb = jnp.take_along_axis(bx, order[:, :, None], axis=1)
    x1, y1, x2, y2 = (sb[..., i] for i in range(4))
    aa = (_NMS_THR / (1.0 + _NMS_THR)) * (
        jnp.maximum(x2 - x1, 0.0) * jnp.maximum(y2 - y1, 0.0))

    padw = ((0, 0), (0, _NPAD - _N))
    pf = lambda a: jnp.pad(a, padw)
    keep = _run_suppress(pf(x1), pf(y1), pf(x2), pf(y2), pf(aa), pf(ss))

    out_sorted = jnp.where(keep[:, :_N] > 0.0, ss, 0.0)
    row = jnp.arange(_P, dtype=jnp.int32)[:, None]
    out = jnp.zeros((_P, _N), jnp.float32).at[row, order].set(out_sorted)
    return out.reshape(_B, _C, _N)


# speedup vs baseline: 1.3481x; 1.3481x over previous
import jax
import jax.numpy as jnp
from jax.experimental import pallas as pl
from jax.experimental.pallas import tpu as pltpu

_NMS_THR = 0.45
_PRE = 0.005
_B, _N, _C = 8, 5000, 4
_P = _B * _C
_T = 128
_NPAD = 5120
_NB = _NPAD // _T
_BIG = 1e30


def _suppress_kernel(x1r, y1r, x2r, y2r, aar, ssr, keepr, mref):
    keepr[:, :] = jnp.ones((_P, _NPAD), jnp.float32)
    lane = jax.lax.broadcasted_iota(jnp.int32, (_P, _T), 1)
    tri = (jax.lax.broadcasted_iota(jnp.int32, (_T, _T), 1)
           > jax.lax.broadcasted_iota(jnp.int32, (_T, _T), 0))[None]

    def outer(k, carry):
        base = k * _T
        bx1 = x1r[:, pl.ds(base, _T)]
        by1 = y1r[:, pl.ds(base, _T)]
        bx2 = x2r[:, pl.ds(base, _T)]
        by2 = y2r[:, pl.ds(base, _T)]
        baa = aar[:, pl.ds(base, _T)]
        bss = ssr[:, pl.ds(base, _T)]
        gaa = jnp.where(bss > 0.0, baa, _BIG)

        iw = jnp.maximum(
            jnp.minimum(bx2[:, :, None], bx2[:, None, :])
            - jnp.maximum(bx1[:, :, None], bx1[:, None, :]), 0.0)
        ih = (jnp.minimum(by2[:, :, None], by2[:, None, :])
              - jnp.maximum(by1[:, :, None], by1[:, None, :]))
        d = iw * ih - (gaa[:, :, None] + baa[:, None, :])
        mref[...] = jnp.where(jnp.logical_and(d > 0.0, tri), 1.0, 0.0)

        def intra(i, bkeep):
            ki = jnp.sum(bkeep * jnp.where(lane == i, 1.0, 0.0),
                         axis=1, keepdims=True)
            row = mref[:, pl.ds(i, 1), :][:, 0, :]
            return jnp.where(
                jnp.logical_and(row > 0.0, ki > 0.0), 0.0, bkeep)

        bkeep = jax.lax.fori_loop(
            0, _T, intra, keepr[:, pl.ds(base, _T)])
        keepr[:, pl.ds(base, _T)] = bkeep

        kaa = jnp.where(bkeep > 0.0, gaa, _BIG)
        ex1 = bx1[:, :, None]
        ey1 = by1[:, :, None]
        ex2 = bx2[:, :, None]
        ey2 = by2[:, :, None]
        eaa = kaa[:, :, None]

        def over_m(m, carry2):
            mb = m * _T
            mx1 = x1r[:, pl.ds(mb, _T)][:, None, :]
            my1 = y1r[:, pl.ds(mb, _T)][:, None, :]
            mx2 = x2r[:, pl.ds(mb, _T)][:, None, :]
            my2 = y2r[:, pl.ds(mb, _T)][:, None, :]
            maa = aar[:, pl.ds(mb, _T)][:, None, :]
            iw = jnp.maximum(jnp.minimum(ex2, mx2) - jnp.maximum(ex1, mx1), 0.0)
            ih = jnp.minimum(ey2, my2) - jnp.maximum(ey1, my1)
            dd = iw * ih - (eaa + maa)
            dj = jnp.max(dd, axis=1)
            mk = keepr[:, pl.ds(mb, _T)]
            keepr[:, pl.ds(mb, _T)] = jnp.where(dj > 0.0, 0.0, mk)
            return carry2

        jax.lax.fori_loop(k + 1, _NB, over_m, 0)
        return carry

    jax.lax.fori_loop(0, _NB, outer, 0)


def _run_suppress(x1, y1, x2, y2, aa, ss, interpret=False):
    return pl.pallas_call(
        _suppress_kernel,
        out_shape=jax.ShapeDtypeStruct((_P, _NPAD), jnp.float32),
        scratch_shapes=[pltpu.VMEM((_P, _T, _T), jnp.float32)],
        interpret=interpret,
    )(x1, y1, x2, y2, aa, ss)


def kernel(bbs, conf):
    s = jnp.where(conf > _PRE, conf, 0.0).reshape(_P, _N)
    order = jnp.argsort(-s, axis=-1)
    ss = jnp.take_along_axis(s, order, axis=-1)
    bx = jnp.broadcast_to(bbs[:, None], (_B, _C, _N, 4)).reshape(_P, _N, 4)
    sb = jnp.take_along_axis(bx, order[:, :, None], axis=1)
    x1, y1, x2, y2 = (sb[..., i] for i in range(4))
    aa = (_NMS_THR / (1.0 + _NMS_THR)) * (
        jnp.maximum(x2 - x1, 0.0) * jnp.maximum(y2 - y1, 0.0))

    padw = ((0, 0), (0, _NPAD - _N))
    pf = lambda a: jnp.pad(a, padw)
    keep = _run_suppress(pf(x1), pf(y1), pf(x2), pf(y2), pf(aa), pf(ss))

    out_sorted = jnp.where(keep[:, :_N] > 0.0, ss, 0.0)
    inv = jnp.argsort(order, axis=-1)
    out = jnp.take_along_axis(out_sorted, inv, axis=-1)
    return out.reshape(_B, _C, _N)


# speedup vs baseline: 1.4630x; 1.0852x over previous
import jax
import jax.numpy as jnp
from jax.experimental import pallas as pl
from jax.experimental.pallas import tpu as pltpu

_NMS_THR = 0.45
_PRE = 0.005
_B, _N, _C = 8, 5000, 4
_P = _B * _C
_T = 128
_NPAD = 5120
_NB = _NPAD // _T
_BIG = 1e30


def _suppress_kernel(x1r, y1r, x2r, y2r, aar, ssr, keepr, mref):
    keepr[:, :] = jnp.ones((_P, _NPAD), jnp.float32)
    tri =(jax.lax.broadcasted_iota(jnp.int32, (_T, _T), 1)
           > jax.lax.broadcasted_iota(jnp.int32, (_T, _T), 0))[None]

    def outer(k, carry):
        base = k * _T
        bx1 = x1r[:, pl.ds(base, _T)]
        by1 = y1r[:, pl.ds(base, _T)]
        bx2 = x2r[:, pl.ds(base, _T)]
        by2 = y2r[:, pl.ds(base, _T)]
        baa = aar[:, pl.ds(base, _T)]
        bss = ssr[:, pl.ds(base, _T)]
        gaa = jnp.where(bss > 0.0, baa, _BIG)

        iw = jnp.maximum(
            jnp.minimum(bx2[:, :, None], bx2[:, None, :])
            - jnp.maximum(bx1[:, :, None], bx1[:, None, :]), 0.0)
        ih = (jnp.minimum(by2[:, :, None], by2[:, None, :])
              - jnp.maximum(by1[:, :, None], by1[:, None, :]))
        d = iw * ih - (gaa[:, :, None] + baa[:, None, :])
        mref[...] = jnp.where(jnp.logical_and(d > 0.0, tri), 1.0, 0.0)

        bkeep = keepr[:, pl.ds(base, _T)]
        for i in range(_T):
            ki = bkeep[:, i:i + 1]
            bkeep = bkeep * (1.0 - mref[:, i, :] * ki)
        keepr[:, pl.ds(base, _T)] = bkeep

        kaa = jnp.where(bkeep > 0.0, gaa, _BIG)
        ex1 = bx1[:, :, None]
        ey1 = by1[:, :, None]
        ex2 = bx2[:, :, None]
        ey2 = by2[:, :, None]
        eaa = kaa[:, :, None]

        def over_m(m, carry2):
            mb = m * _T
            mx1 = x1r[:, pl.ds(mb, _T)][:, None, :]
            my1 = y1r[:, pl.ds(mb, _T)][:, None, :]
            mx2 = x2r[:, pl.ds(mb, _T)][:, None, :]
            my2 = y2r[:, pl.ds(mb, _T)][:, None, :]
            maa = aar[:, pl.ds(mb, _T)][:, None, :]
            iw = jnp.maximum(jnp.minimum(ex2, mx2) - jnp.maximum(ex1, mx1), 0.0)
            ih = jnp.minimum(ey2, my2) - jnp.maximum(ey1, my1)
            dd = iw * ih - (eaa + maa)
            dj = jnp.max(dd, axis=1)
            mk = keepr[:, pl.ds(mb, _T)]
            keepr[:, pl.ds(mb, _T)] = jnp.where(dj > 0.0, 0.0, mk)
            return carry2

        jax.lax.fori_loop(k + 1, _NB, over_m, 0)
        return carry

    jax.lax.fori_loop(0, _NB, outer, 0)


def _run_suppress(x1, y1, x2, y2, aa, ss, interpret=False):
    return pl.pallas_call(
        _suppress_kernel,
        out_shape=jax.ShapeDtypeStruct((_P, _NPAD), jnp.float32),
        scratch_shapes=[pltpu.VMEM((_P, _T, _T), jnp.float32)],
        interpret=interpret,
    )(x1, y1, x2, y2, aa, ss)


def kernel(bbs, conf):
    s = jnp.where(conf > _PRE, conf, 0.0).reshape(_P, _N)
    order = jnp.argsort(-s, axis=-1)
    ss = jnp.take_along_axis(s, order, axis=-1)
    bx = jnp.broadcast_to(bbs[:, None], (_B, _C, _N, 4)).reshape(_P, _N, 4)
    sb = jnp.take_along_axis(bx, order[:, :, None], axis=1)
    x1, y1, x2, y2 = (sb[..., i] for i in range(4))
    aa = (_NMS_THR / (1.0 + _NMS_THR)) * (
        jnp.maximum(x2 - x1, 0.0) * jnp.maximum(y2 - y1, 0.0))

    padw = ((0, 0), (0, _NPAD - _N))
    pf = lambda a: jnp.pad(a, padw)
    keep = _run_suppress(pf(x1), pf(y1), pf(x2), pf(y2), pf(aa), pf(ss))

    out_sorted = jnp.where(keep[:, :_N] > 0.0, ss, 0.0)
    inv = jnp.argsort(order, axis=-1)
    out = jnp.take_along_axis(out_sorted, inv, axis=-1)
    return out.reshape(_B, _C, _N)
